# baseline (speedup 1.0000x reference)
import jax
import jax.numpy as jnp
from jax import lax
from jax.experimental import pallas as pl
from jax.experimental.pallas import tpu as pltpu
from jax.experimental.pallas import tpu_sc as plsc

_B = 128
_N = 32768
_MAX_K = 64
_MIN_K = 8
_CONF = 0.9

_L = 16
_NW = 32
_RPW = _B // _NW
_NV = _N // _L
_NCHUNK = 64
_CV = _NV // _NCHUNK
_NVMAX = (_N // _L) + 1
_NSMAX = (_NVMAX + _L - 1) // _L
_VBUF = _NSMAX * _L * _L + _L
_BIG = 2**30


def _topk_body(probs_hbm, out_val_hbm, out_idx_hbm, out_eff_hbm,
               row_v, cmx_buf, val_buf, idx_buf, smax_buf,
               out_val_v, out_idx_v, eff_v, dma_sem):
    cid = lax.axis_index("c")
    sid = lax.axis_index("s")
    wid = sid * 2 + cid

    iota = lax.iota(jnp.int32, _L)
    effv = jnp.zeros((_L,), jnp.int32)

    cp = pltpu.async_copy(probs_hbm.at[wid * _RPW], row_v, dma_sem)

    for r in range(_RPW):
        cp.wait()

        def cmx_step(g, unused):
            acc = jnp.full((_L,), -1.0, jnp.float32)
            for ci in range(_L):
                base = (g * _L + ci) * 256
                a = [row_v[pl.ds(base + k * _L, _L)] for k in range(4)]
                for k in range(4, 16):
                    a[k % 4] = jnp.maximum(
                        a[k % 4], row_v[pl.ds(base + k * _L, _L)])
                am = jnp.maximum(jnp.maximum(a[0], a[1]),
                                 jnp.maximum(a[2], a[3]))
                acc = jnp.where(iota == ci, jnp.max(am), acc)
            cmx_buf[pl.ds(g * _L, _L)] = acc
            return unused

        lax.fori_loop(0, 8, cmx_step, jnp.int32(0))

        cmb = [plsc.bitcast(cmx_buf[pl.ds(g * _L, _L)], jnp.int32)
               for g in range(8)]

        def bs_step(it, lohi):
            lo, hi = lohi
            mid = (lo + hi) // 2
            cv = plsc.all_reduce_population_count(cmb[0] >= mid)
            for g in range(1, 8):
                cv = cv + plsc.all_reduce_population_count(cmb[g] >= mid)
            ok = cv[0] >= _MAX_K
            return (jnp.where(ok, mid, lo), jnp.where(ok, hi, mid))

        lo, _ = lax.fori_loop(0, 24, bs_step,
                              (jnp.int32(0), jnp.int32(0x3F800000)))
        t = lax.bitcast_convert_type(lo, jnp.float32)

        def filt_step(i, wp):
            base = i * 16 * _L
            v = [row_v[pl.ds(base + k * _L, _L)] for k in range(16)]
            m = [vk >= t for vk in v]
            c = [plsc.all_reduce_population_count(mk)[0] for mk in m]
            w = wp
            for k in range(16):
                plsc.store_compressed(idx_buf.at[pl.ds(w, _L)],
                                      base + k * _L + iota, mask=m[k])
                w = w + c[k]
            return w

        s = lax.fori_loop(0, _NV // 16, filt_step, jnp.int32(0))

        nv = (s + _L - 1) // _L

        def gb_step(k, unused):
            idxv = idx_buf[pl.ds(k * _L, _L)]
            idxc = jnp.clip(idxv, 0, _N - 1)
            val_buf[pl.ds(k * _L, _L)] = plsc.load_gather(row_v, [idxc])
            return unused

        lax.fori_loop(0, nv, gb_step, jnp.int32(0))

        val_buf[pl.ds(s, _L)] = jnp.full((_L,), -1.0, jnp.float32)

        if r + 1 < _RPW:
            cp = pltpu.async_copy(probs_hbm.at[wid * _RPW + r + 1],
                                  row_v, dma_sem)

        ns = (nv + _L - 1) // _L

        def sum_step(k, unused):
            acc = jnp.full((_L,), -1.0, jnp.float32)
            for l in range(_L):
                vv = val_buf[pl.ds((k * _L + l) * _L, _L)]
                mx = jnp.where(k * _L + l < nv, jnp.max(vv),
                               jnp.float32(-1.0))
                acc = jnp.where(iota == l, mx, acc)
            smax_buf[pl.ds(k * _L, _L)] = acc
            return unused

        lax.fori_loop(0, ns, sum_step, jnp.int32(0))

        def sel_step(j, carry):
            csum, cnt = carry

            def sscan(k, bc):
                bestv, besti = bc
                v = smax_buf[pl.ds(k * _L, _L)]
                m = v > bestv
                return (jnp.where(m, v, bestv), jnp.where(m, k, besti))

            bestv, besti = lax.fori_loop(
                0, ns, sscan,
                (jnp.full((_L,), -2.0, jnp.float32),
                 jnp.zeros((_L,), jnp.int32)))

            mval = jnp.max(bestv)
            vno = jnp.min(jnp.where(bestv == mval, besti * _L + iota,
                                    _BIG))

            vv = val_buf[pl.ds(vno * _L, _L)]
            tie = vv == mval
            lane = jnp.min(jnp.where(tie, iota, _BIG))
            ivec = idx_buf[pl.ds(vno * _L, _L)]
            oidx = jnp.min(jnp.where(tie, ivec, _BIG))

            vv2 = jnp.where(iota == lane, -1.0, vv)
            val_buf[pl.ds(vno * _L, _L)] = vv2
            newm = jnp.max(vv2)
            sb = (vno // _L) * _L
            sv = smax_buf[pl.ds(sb, _L)]
            smax_buf[pl.ds(sb, _L)] = jnp.where(iota == vno - sb, newm, sv)

            jbase = (j // _L) * _L
            jm = iota == (j - jbase)
            wv = out_val_v.at[r][pl.ds(jbase, _L)]
            out_val_v.at[r][pl.ds(jbase, _L)] = jnp.where(jm, mval, wv)
            wi = out_idx_v.at[r][pl.ds(jbase, _L)]
            out_idx_v.at[r][pl.ds(jbase, _L)] = jnp.where(jm, oidx, wi)

            csum = csum + mval
            cnt = cnt + jnp.where(csum < _CONF, 1, 0).astype(jnp.int32)
            return (csum, cnt)

        _, cnt = lax.fori_loop(0, _MAX_K, sel_step,
                               (jnp.float32(0.0), jnp.int32(0)))

        eff = jnp.clip(cnt + 1, _MIN_K, _MAX_K)
        effv = jnp.where(iota == r, eff, effv)

    eff_v[...] = effv
    pltpu.sync_copy(out_val_v, out_val_hbm.at[pl.ds(wid * _RPW, _RPW)])
    pltpu.sync_copy(out_idx_v, out_idx_hbm.at[pl.ds(wid * _RPW, _RPW)])
    pltpu.sync_copy(eff_v, out_eff_hbm.at[wid])


@jax.jit
def _sc_topk(probs):
    mesh = plsc.VectorSubcoreMesh(core_axis_name="c", subcore_axis_name="s")
    fn = pl.kernel(
        _topk_body,
        out_type=[
            jax.ShapeDtypeStruct((_B, _MAX_K), jnp.float32),
            jax.ShapeDtypeStruct((_B, _MAX_K), jnp.int32),
            jax.ShapeDtypeStruct((_NW, _L), jnp.int32),
        ],
        mesh=mesh,
        compiler_params=pltpu.CompilerParams(needs_layout_passes=False),
        scratch_types=[
            pltpu.VMEM((_N,), jnp.float32),
            pltpu.VMEM((8 * _L,), jnp.float32),
            pltpu.VMEM((_VBUF,), jnp.float32),
            pltpu.VMEM((_N + _L,), jnp.int32),
            pltpu.VMEM((_NSMAX * _L,), jnp.float32),
            pltpu.VMEM((_RPW, _MAX_K), jnp.float32),
            pltpu.VMEM((_RPW, _MAX_K), jnp.int32),
            pltpu.VMEM((_L,), jnp.int32),
            pltpu.SemaphoreType.DMA,
        ],
    )
    return fn(probs)


def kernel(probs):
    vals, idx, eff = _sc_topk(probs)
    return (vals, idx.astype(jnp.int64),
            eff[:, :_RPW].reshape(_B))


# speedup vs baseline: 1.3425x; 1.3425x over previous
import jax
import jax.numpy as jnp
from jax import lax
from jax.experimental import pallas as pl
from jax.experimental.pallas import tpu as pltpu
from jax.experimental.pallas import tpu_sc as plsc

_B = 128
_N = 32768
_MAX_K = 64
_MIN_K = 8
_CONF = 0.9

_L = 16
_NW = 32
_RPW = _B // _NW
_NV = _N // _L
_NCHUNK = 64
_CV = _NV // _NCHUNK
_NVMAX = (_N // _L) + 1
_NSMAX = (_NVMAX + _L - 1) // _L
_VBUF = _NSMAX * _L * _L + _L
_BIG = 2**30


def _topk_body(probs_hbm, out_val_hbm, out_idx_hbm, out_eff_hbm,
               row_v, val_buf, idx_buf, smax_buf,
               out_val_v, out_idx_v, eff_v, dma_sem):
    cid = lax.axis_index("c")
    sid = lax.axis_index("s")
    wid = sid * 2 + cid

    iota = lax.iota(jnp.int32, _L)
    effv = jnp.zeros((_L,), jnp.int32)

    cp = pltpu.async_copy(probs_hbm.at[wid * _RPW], row_v, dma_sem)

    for r in range(_RPW):
        cp.wait()

        def chunk_step(c2, t):
            tt = t
            for h in range(2):
                base = (c2 * 2 + h) * (_CV * _L)
                a = [row_v[pl.ds(base + k * _L, _L)] for k in range(4)]
                for k in range(4, _CV):
                    a[k % 4] = jnp.maximum(
                        a[k % 4], row_v[pl.ds(base + k * _L, _L)])
                am = jnp.maximum(jnp.maximum(a[0], a[1]),
                                 jnp.maximum(a[2], a[3]))
                tt = jnp.minimum(tt, jnp.max(am))
            return tt

        t = lax.fori_loop(0, _NCHUNK // 2, chunk_step,
                          jnp.float32(jnp.inf))

        def filt_step(i, wp):
            base = i * 16 * _L
            v = [row_v[pl.ds(base + k * _L, _L)] for k in range(16)]
            m = [vk >= t for vk in v]
            c = [plsc.all_reduce_population_count(mk)[0] for mk in m]
            w = wp
            for k in range(16):
                plsc.store_compressed(idx_buf.at[pl.ds(w, _L)],
                                      base + k * _L + iota, mask=m[k])
                w = w + c[k]
            return w

        s = lax.fori_loop(0, _NV // 16, filt_step, jnp.int32(0))

        nv = (s + _L - 1) // _L

        def gb_step(k, unused):
            idxv = idx_buf[pl.ds(k * _L, _L)]
            idxc = jnp.clip(idxv, 0, _N - 1)
            val_buf[pl.ds(k * _L, _L)] = plsc.load_gather(row_v, [idxc])
            return unused

        lax.fori_loop(0, nv, gb_step, jnp.int32(0))

        val_buf[pl.ds(s, _L)] = jnp.full((_L,), -1.0, jnp.float32)

        if r + 1 < _RPW:
            cp = pltpu.async_copy(probs_hbm.at[wid * _RPW + r + 1],
                                  row_v, dma_sem)

        ns = (nv + _L - 1) // _L

        def sum_step(k, unused):
            acc = jnp.full((_L,), -1.0, jnp.float32)
            for l in range(_L):
                vv = val_buf[pl.ds((k * _L + l) * _L, _L)]
                mx = jnp.where(k * _L + l < nv, jnp.max(vv),
                               jnp.float32(-1.0))
                acc = jnp.where(iota == l, mx, acc)
            smax_buf[pl.ds(k * _L, _L)] = acc
            return unused

        lax.fori_loop(0, ns, sum_step, jnp.int32(0))

        def sel_step(j, carry):
            csum, cnt = carry

            def sscan(k, bc):
                bestv, besti = bc
                v = smax_buf[pl.ds(k * _L, _L)]
                m = v > bestv
                return (jnp.where(m, v, bestv), jnp.where(m, k, besti))

            bestv, besti = lax.fori_loop(
                0, ns, sscan,
                (jnp.full((_L,), -2.0, jnp.float32),
                 jnp.zeros((_L,), jnp.int32)))

            mval = jnp.max(bestv)
            vno = jnp.min(jnp.where(bestv == mval, besti * _L + iota,
                                    _BIG))

            vv = val_buf[pl.ds(vno * _L, _L)]
            tie = vv == mval
            lane = jnp.min(jnp.where(tie, iota, _BIG))
            ivec = idx_buf[pl.ds(vno * _L, _L)]
            oidx = jnp.min(jnp.where(tie, ivec, _BIG))

            vv2 = jnp.where(iota == lane, -1.0, vv)
            val_buf[pl.ds(vno * _L, _L)] = vv2
            newm = jnp.max(vv2)
            sb = (vno // _L) * _L
            sv = smax_buf[pl.ds(sb, _L)]
            smax_buf[pl.ds(sb, _L)] = jnp.where(iota == vno - sb, newm, sv)

            jbase = (j // _L) * _L
            jm = iota == (j - jbase)
            wv = out_val_v.at[r][pl.ds(jbase, _L)]
            out_val_v.at[r][pl.ds(jbase, _L)] = jnp.where(jm, mval, wv)
            wi = out_idx_v.at[r][pl.ds(jbase, _L)]
            out_idx_v.at[r][pl.ds(jbase, _L)] = jnp.where(jm, oidx, wi)

            csum = csum + mval
            cnt = cnt + jnp.where(csum < _CONF, 1, 0).astype(jnp.int32)
            return (csum, cnt)

        _, cnt = lax.fori_loop(0, _MAX_K, sel_step,
                               (jnp.float32(0.0), jnp.int32(0)))

        eff = jnp.clip(cnt + 1, _MIN_K, _MAX_K)
        effv = jnp.where(iota == r, eff, effv)

    eff_v[...] = effv
    pltpu.sync_copy(out_val_v, out_val_hbm.at[pl.ds(wid * _RPW, _RPW)])
    pltpu.sync_copy(out_idx_v, out_idx_hbm.at[pl.ds(wid * _RPW, _RPW)])
    pltpu.sync_copy(eff_v, out_eff_hbm.at[wid])


@jax.jit
def _sc_topk(probs):
    mesh = plsc.VectorSubcoreMesh(core_axis_name="c", subcore_axis_name="s")
    fn = pl.kernel(
        _topk_body,
        out_type=[
            jax.ShapeDtypeStruct((_B, _MAX_K), jnp.float32),
            jax.ShapeDtypeStruct((_B, _MAX_K), jnp.int32),
            jax.ShapeDtypeStruct((_NW, _L), jnp.int32),
        ],
        mesh=mesh,
        compiler_params=pltpu.CompilerParams(needs_layout_passes=False),
        scratch_types=[
            pltpu.VMEM((_N,), jnp.float32),
            pltpu.VMEM((_VBUF,), jnp.float32),
            pltpu.VMEM((_N + _L,), jnp.int32),
            pltpu.VMEM((_NSMAX * _L,), jnp.float32),
            pltpu.VMEM((_RPW, _MAX_K), jnp.float32),
            pltpu.VMEM((_RPW, _MAX_K), jnp.int32),
            pltpu.VMEM((_L,), jnp.int32),
            pltpu.SemaphoreType.DMA,
        ],
    )
    return fn(probs)


def kernel(probs):
    vals, idx, eff = _sc_topk(probs)
    return (vals, idx.astype(jnp.int64),
            eff[:, :_RPW].reshape(_B))
